# Initial kernel scaffold; baseline (speedup 1.0000x reference)
import jax
import jax.numpy as jnp
from jax.experimental import pallas as pl


def kernel(x, ds_w1, ds_b1, ds_w2, ds_b2, pool_w, ds_wr, ds_br, ds_lng, ds_lnb, router_w, router_b, e_w1, e_b1, e_w2, e_b2, vp_w, vp_b, sp_w, sp_b, out_w, out_b, bk_scale, up_w1, up_b1, up_lng, up_lnb, up_w2, up_b2, pos_embed, rf_lng, rf_lnb, rf_w1, rf_b1, rf_w2, rf_b2, scale_low, scale_ref):
    raise NotImplementedError("write your pallas kernel here")


# speedup vs baseline: 1.2991x; 1.2991x over previous
import functools

import jax
import jax.numpy as jnp
from jax.experimental import pallas as pl
from jax.experimental.pallas import tpu as pltpu

D = 768
N = 2048
ND = N // 2
E = 8
H = 768
TS = 0.6
B = 2
NTOK = B * ND
K_KEEP = max(1, int(ND * (1.0 - TS)))


def _gelu(x):
    return 0.5 * x * (1.0 + jax.lax.erf(x * (2.0 ** -0.5)))


def _ln(x, g, b, eps=1e-5):
    m = jnp.mean(x, axis=-1, keepdims=True)
    v = jnp.mean((x - m) ** 2, axis=-1, keepdims=True)
    return (x - m) * jax.lax.rsqrt(v + eps) * g + b


def _dot(a, b):
    return jax.lax.dot_general(a, b, (((1,), (0,)), ((), ())),
                               preferred_element_type=jnp.float32)


def _a_body(xf, xe, xo, poolw, w1, b1, w2r, b2, wr, br, lng, lnb,
            imp_out, xd_out):
    a1 = jnp.maximum(_dot(xf[...], w1[...]) + b1[...], 0.0)
    imp_out[...] = (jnp.sum(a1 * w2r[...], axis=-1, keepdims=True)
                    + b2[0, 0])
    pw = poolw[...]
    pm = jnp.max(pw, axis=-1, keepdims=True)
    pe = jnp.exp(pw - pm)
    ps = pe / jnp.sum(pe, axis=-1, keepdims=True)
    xd0 = xe[...] * ps[:, 0:1] + xo[...] * ps[:, 1:2]
    h = _dot(xd0, wr[...]) + br[...]
    xd_out[...] = _gelu(_ln(h, lng[...], lnb[...]))


def _stage_a(xf, xe, xo, poolw, w1, b1, w2r, b2, wr, br, lng, lnb):
    nblk = 16
    tb = (B * N) // nblk
    db = tb // 2
    return pl.pallas_call(
        _a_body,
        grid=(nblk,),
        in_specs=[
            pl.BlockSpec((tb, D), lambda i: (i, 0)),
            pl.BlockSpec((db, D), lambda i: (i, 0)),
            pl.BlockSpec((db, D), lambda i: (i, 0)),
            pl.BlockSpec((db, 2), lambda i: (i % (ND // db), 0)),
            pl.BlockSpec((D, D // 2), lambda i: (0, 0)),
            pl.BlockSpec((1, D // 2), lambda i: (0, 0)),
            pl.BlockSpec((1, D // 2), lambda i: (0, 0)),
            pl.BlockSpec((1, 1), lambda i: (0, 0)),
            pl.BlockSpec((D, D), lambda i: (0, 0)),
            pl.BlockSpec((1, D), lambda i: (0, 0)),
            pl.BlockSpec((1, D), lambda i: (0, 0)),
            pl.BlockSpec((1, D), lambda i: (0, 0)),
        ],
        out_specs=[
            pl.BlockSpec((tb, 1), lambda i: (i, 0)),
            pl.BlockSpec((db, D), lambda i: (i, 0)),
        ],
        out_shape=[
            jax.ShapeDtypeStruct((B * N, 1), jnp.float32),
            jax.ShapeDtypeStruct((NTOK, D), jnp.float32),
        ],
    )(xf, xe, xo, poolw, w1, b1, w2r, b2, wr, br, lng, lnb)


def _b_body(xd, rw, rb, spwr, spb, wgt_out, mask_out, sp_out):
    x = xd[...]
    logits = _dot(x, rw[...]) + rb[...]
    lm = jnp.max(logits, axis=-1, keepdims=True)
    eg = jnp.exp(logits - lm)
    gates = eg / jnp.sum(eg, axis=-1, keepdims=True)
    gmax = jnp.max(gates, axis=-1, keepdims=True)
    lane = jax.lax.broadcasted_iota(jnp.int32, (NTOK, E), 1)
    eid = jnp.min(jnp.where(gates >= gmax, lane, E + 1), axis=-1,
                  keepdims=True)
    wgt_out[...] = jnp.where(lane == eid, gmax, 0.0)

    score = jnp.sum(x * spwr[...], axis=-1, keepdims=True) + spb[0, 0]
    row = jax.lax.broadcasted_iota(jnp.int32, (NTOK, 1), 0)
    bmask0 = (row < ND).astype(jnp.float32)
    bmask1 = 1.0 - bmask0
    lo = jnp.full((NTOK, 1), -1e30, jnp.float32)
    hi = jnp.full((NTOK, 1), 1e30, jnp.float32)

    def body(_, carry):
        lo, hi = carry
        mid = 0.5 * (lo + hi)
        ge = (score >= mid).astype(jnp.float32)
        c0 = jnp.sum(ge * bmask0)
        c1 = jnp.sum(ge * bmask1)
        cnt = bmask0 * c0 + bmask1 * c1
        keep = cnt >= K_KEEP
        return (jnp.where(keep, mid, lo), jnp.where(keep, hi, mid))

    lo, hi = jax.lax.fori_loop(0, 120, body, (lo, hi))
    mask = (score >= lo).astype(jnp.float32)
    mask_out[...] = mask
    sp_out[...] = jnp.reshape(1.0 - jnp.sum(mask) / float(NTOK), (1, 1))


def _stage_b(xd, rw, rb, spwr, spb):
    return pl.pallas_call(
        _b_body,
        in_specs=[pl.BlockSpec(a.shape, lambda: (0, 0))
                  for a in (xd, rw, rb, spwr, spb)],
        out_specs=[
            pl.BlockSpec((NTOK, E), lambda: (0, 0)),
            pl.BlockSpec((NTOK, 1), lambda: (0, 0)),
            pl.BlockSpec((1, 1), lambda: (0, 0)),
        ],
        out_shape=[
            jax.ShapeDtypeStruct((NTOK, E), jnp.float32),
            jax.ShapeDtypeStruct((NTOK, 1), jnp.float32),
            jax.ShapeDtypeStruct((1, 1), jnp.float32),
        ],
    )(xd, rw, rb, spwr, spb)


def _c_body(xd, w1, b1, w2, b2, wgt, out):
    e = pl.program_id(1)
    lane = jax.lax.broadcasted_iota(jnp.int32, wgt.shape, 1)
    col = jnp.sum(jnp.where(lane == e, wgt[...], 0.0), axis=-1,
                  keepdims=True)
    h = _gelu(_dot(xd[...], w1[0]) + b1[0])
    o = _dot(h, w2[0]) + b2[0]

    @pl.when(e == 0)
    def _():
        out[...] = col * o

    @pl.when(e > 0)
    def _():
        out[...] += col * o


def _stage_c(xd, w1, b1, w2, b2, wgt):
    tb = 256
    return pl.pallas_call(
        _c_body,
        grid=(NTOK // tb, E),
        in_specs=[
            pl.BlockSpec((tb, D), lambda t, e: (t, 0)),
            pl.BlockSpec((1, D, H), lambda t, e: (e, 0, 0)),
            pl.BlockSpec((1, 1, H), lambda t, e: (e, 0, 0)),
            pl.BlockSpec((1, H, D), lambda t, e: (e, 0, 0)),
            pl.BlockSpec((1, 1, D), lambda t, e: (e, 0, 0)),
            pl.BlockSpec((tb, E), lambda t, e: (t, 0)),
        ],
        out_specs=pl.BlockSpec((tb, D), lambda t, e: (t, 0)),
        out_shape=jax.ShapeDtypeStruct((NTOK, D), jnp.float32),
    )(xd, w1, b1, w2, b2, wgt)


def _d_body(ffn, msk, xpk, vpwr, vpb, outw, outb, bks, uw1, ub1, ulng, ulnb,
            uw2, ub2, pospk, rlng, rlnb, rw1, rb1, rw2, rb2, sl, sr, out):
    f = ffn[...]
    v = jnp.clip(jnp.sum(f * vpwr[...], axis=-1, keepdims=True) + vpb[0, 0],
                 -3.0, 3.0)
    den = v * v + 1.0
    m = msk[...]
    f0 = jnp.clip((v / den) * m, -10.0, 10.0)
    f1 = jnp.clip((-1.0 / den) * m, -10.0, 10.0)
    spec = f0 * outw[0:1, :] + f1 * outw[1:2, :] + outb[...]
    x_low = f + bks[0, 0] * spec
    t1 = _dot(x_low, uw1[...]) + ub1[...]
    t1 = _gelu(_ln(t1, ulng[...], ulnb[...]))
    xt = _dot(t1, uw2[...]) + ub2[...]
    xu = xt + pospk[...]
    res = xpk[...] + sl[0, 0] * xu
    g = rlng[...]
    b = rlnb[...]
    for half in range(2):
        s = slice(half * D, (half + 1) * D)
        u = xu[:, s]
        n = _ln(u, g, b)
        r = _dot(_gelu(_dot(n, rw1[...]) + rb1[...]), rw2[...]) + rb2[...]
        out[:, s] = res[:, s] + sr[0, 0] * r


def _stage_d(ffn, msk, xpk, vpwr, vpb, outw, outb, bks, uw1, ub1, ulng, ulnb,
             uw2, ub2, pospk, rlng, rlnb, rw1, rb1, rw2, rb2, sl, sr):
    tb = 256
    full = lambda a: pl.BlockSpec(a.shape, lambda t: (0,) * a.ndim)
    return pl.pallas_call(
        _d_body,
        grid=(NTOK // tb,),
        in_specs=[
            pl.BlockSpec((tb, D), lambda t: (t, 0)),
            pl.BlockSpec((tb, 1), lambda t: (t, 0)),
            pl.BlockSpec((tb, 2 * D), lambda t: (t, 0)),
            full(vpwr), full(vpb), full(outw), full(outb), full(bks),
            full(uw1), full(ub1), full(ulng), full(ulnb), full(uw2),
            full(ub2), full(pospk), full(rlng), full(rlnb), full(rw1),
            full(rb1), full(rw2), full(rb2), full(sl), full(sr),
        ],
        out_specs=pl.BlockSpec((tb, 2 * D), lambda t: (t, 0)),
        out_shape=jax.ShapeDtypeStruct((NTOK, 2 * D), jnp.float32),
    )(ffn, msk, xpk, vpwr, vpb, outw, outb, bks, uw1, ub1, ulng, ulnb,
      uw2, ub2, pospk, rlng, rlnb, rw1, rb1, rw2, rb2, sl, sr)


def kernel(x, ds_w1, ds_b1, ds_w2, ds_b2, pool_w, ds_wr, ds_br, ds_lng,
           ds_lnb, router_w, router_b, e_w1, e_b1, e_w2, e_b2, vp_w, vp_b,
           sp_w, sp_b, out_w, out_b, bk_scale, up_w1, up_b1, up_lng, up_lnb,
           up_w2, up_b2, pos_embed, rf_lng, rf_lnb, rf_w1, rf_b1, rf_w2,
           rf_b2, scale_low, scale_ref):
    r1 = lambda a: a.reshape(1, -1)
    s11 = lambda a: a.reshape(1, 1)
    xf = x.reshape(B * N, D)
    xe = x[:, 0::2, :].reshape(NTOK, D)
    xo = x[:, 1::2, :].reshape(NTOK, D)

    imp_c, xd = _stage_a(xf, xe, xo, pool_w, ds_w1, r1(ds_b1), r1(ds_w2),
                         s11(ds_b2), ds_wr, r1(ds_br), r1(ds_lng),
                         r1(ds_lnb))
    wgt, mask, sp = _stage_b(xd, router_w, r1(router_b), r1(sp_w),
                             s11(sp_b))
    ffn = _stage_c(xd, e_w1, e_b1.reshape(E, 1, H), e_w2,
                   e_b2.reshape(E, 1, D), wgt)
    xpk = x.reshape(NTOK, 2 * D)
    out_pk = _stage_d(
        ffn, mask, xpk, r1(vp_w), s11(vp_b), out_w, r1(out_b),
        s11(bk_scale), up_w1, r1(up_b1), r1(up_lng), r1(up_lnb), up_w2,
        r1(up_b2), pos_embed.reshape(1, 2 * D), r1(rf_lng), r1(rf_lnb),
        rf_w1, r1(rf_b1), rf_w2, r1(rf_b2), s11(scale_low), s11(scale_ref))

    out = out_pk.reshape(B, N, D)
    imp = imp_c.reshape(B, N)
    return out, imp, sp[0, 0]
